# baseline (speedup 1.0000x reference)
import functools

import jax
import jax.numpy as jnp
from jax import lax
from jax.experimental import pallas as pl
from jax.experimental.pallas import tpu as pltpu
from jax.experimental.pallas import tpu_sc as plsc

N = 100000
E = 1600000
B = 128
NUM_SHAPES = 16
NUM_COLORS = 16
NUM_CLASSES = 8
EMB = 16
HID = 32

BN = 1024
NPAD = 100352
NB = NPAD // BN
ROWS_PT = NPAD // 16

NSUB = 16
CH = 128
SUP = 4
NSUP = 196
EPT = NSUP * SUP * CH
EP = EPT * NSUB
ER = EP // (SUP * CH)

_F32 = jnp.float32
_I32 = jnp.int32


def _make_edge_pass(with_deg: bool):
    outs = [jax.ShapeDtypeStruct((2, NPAD, 16), _F32)]
    scratch = [
        pltpu.VMEM((2, SUP, CH), _I32),
        pltpu.VMEM((2, SUP, CH), _I32),
        pltpu.VMEM((2, SUP, CH, 16), _F32),
        pltpu.VMEM_SHARED((NPAD, 16), _F32),
        pltpu.SemaphoreType.DMA,
        pltpu.SemaphoreType.DMA,
    ]
    if with_deg:
        outs.append(jax.ShapeDtypeStruct((2, NPAD), _F32))
        scratch += [
            pltpu.VMEM((CH,), _F32),
            pltpu.VMEM_SHARED((NPAD,), _F32),
            pltpu.SemaphoreType.DMA,
        ]

    def body(*refs):
        if with_deg:
            (src_h, dst_h, m_h, zc_h, zd_h, s_out, deg_out,
             srcb, dstb, rows, acc_sh, sem_g, sem_s,
             ones_v, deg_sh, sem_d) = refs
        else:
            (src_h, dst_h, m_h, zc_h, s_out,
             srcb, dstb, rows, acc_sh, sem_g, sem_s) = refs
        c = lax.axis_index("c")
        s = lax.axis_index("s")
        r0 = s * ROWS_PT

        pltpu.sync_copy(zc_h, acc_sh.at[pl.ds(r0, ROWS_PT)])
        if with_deg:
            pltpu.sync_copy(zd_h, deg_sh.at[pl.ds(r0, ROWS_PT)])
            for j in range(CH // 16):
                ones_v[pl.ds(j * 16, 16)] = jnp.ones((16,), _F32)
        plsc.subcore_barrier()

        crow0 = s * NSUP
        mt = m_h.at[c]
        pltpu.sync_copy(src_h.at[crow0], srcb.at[0])
        pltpu.sync_copy(dst_h.at[crow0], dstb.at[0])

        def drain_scatters(b):
            for j in range(SUP):
                pltpu.make_async_copy(mt.at[pl.ds(j * CH, CH)],
                                      rows.at[b, j], sem_s).wait()
            if with_deg:
                for _ in range(SUP // 2):
                    pltpu.make_async_copy(zd_h.at[pl.ds(0, CH)],
                                          ones_v, sem_d).wait()

        def sup_body(sup, carry):
            b = lax.rem(sup, 2)
            gs = [pltpu.async_copy(mt.at[srcb.at[b, j]], rows.at[b, j], sem_g)
                  for j in range(SUP)]

            @pl.when(sup > 0)
            def _():
                drain_scatters(1 - b)

            @pl.when(sup + 1 < NSUP)
            def _():
                nb = 1 - b
                nrow = crow0 + sup + 1
                pltpu.sync_copy(src_h.at[nrow], srcb.at[nb])
                pltpu.sync_copy(dst_h.at[nrow], dstb.at[nb])

            if with_deg:
                @pl.when(c == 0)
                def _():
                    for j in range(SUP // 2):
                        pltpu.async_copy(ones_v, deg_sh.at[dstb.at[b, j]],
                                         sem_d, add=True)

                @pl.when(c != 0)
                def _():
                    for j in range(SUP // 2, SUP):
                        pltpu.async_copy(ones_v, deg_sh.at[dstb.at[b, j]],
                                         sem_d, add=True)

            for j in range(SUP):
                gs[j].wait()
                pltpu.async_copy(rows.at[b, j], acc_sh.at[dstb.at[b, j]],
                                 sem_s, add=True)
            return carry

        lax.fori_loop(0, NSUP, sup_body, 0)
        drain_scatters((NSUP - 1) % 2)

        plsc.subcore_barrier()
        pltpu.sync_copy(acc_sh.at[pl.ds(r0, ROWS_PT)],
                        s_out.at[c].at[pl.ds(r0, ROWS_PT)])
        if with_deg:
            pltpu.sync_copy(deg_sh.at[pl.ds(r0, ROWS_PT)],
                            deg_out.at[c].at[pl.ds(r0, ROWS_PT)])

    mesh = plsc.VectorSubcoreMesh(core_axis_name="c", subcore_axis_name="s")
    return pl.kernel(body, out_type=tuple(outs), mesh=mesh,
                     scratch_types=scratch,
                     compiler_params=pltpu.CompilerParams(
                         use_tc_tiling_on_sc=False))


def _oh16(idx_col):
    return (idx_col == lax.broadcasted_iota(_I32, (BN, 16), 1)).astype(_F32)


def _matT(a, w):
    return lax.dot_general(a, w, (((1,), (1,)), ((), ())),
                           preferred_element_type=_F32)


def _prep_body(x_ref, se_ref, ce_ref, w1l_ref, w1r_ref, m_ref, p_ref):
    ohs = _oh16(x_ref[:, 0:1])
    ohc = _oh16(x_ref[:, 1:2])
    h0 = jnp.concatenate(
        [jnp.dot(ohs, se_ref[...], preferred_element_type=_F32),
         jnp.dot(ohc, ce_ref[...], preferred_element_type=_F32)], axis=1)
    m = _matT(h0, w1l_ref[...])
    m_ref[0] = m[:, :16]
    m_ref[1] = m[:, 16:]
    p_ref[...] = _matT(h0, w1r_ref[...])


def _mid_body(lo_ref, hi_ref, degt_ref, p0_ref, b1_ref, w2l_ref, w2r_ref,
              m_ref, p1_ref, rdeg_ref):
    deg = degt_ref[:, 0:1] + degt_ref[:, 1:2]
    rdeg = 1.0 / jnp.maximum(deg, 1.0)
    sfull = jnp.concatenate([lo_ref[0], hi_ref[0]], axis=1)
    h1 = jnp.maximum(sfull * rdeg + b1_ref[...] + p0_ref[...], 0.0)
    m1 = _matT(h1, w2l_ref[...])
    m_ref[0] = m1[:, :16]
    m_ref[1] = m1[:, 16:]
    p1_ref[...] = _matT(h1, w2r_ref[...])
    rdeg_ref[...] = rdeg


def _final_body(lo_ref, hi_ref, rdeg_ref, p1_ref, b2_ref, batch_ref,
                clsw_ref, clsb_ref, out_ref, acc):
    i = pl.program_id(0)

    @pl.when(i == 0)
    def _():
        acc[...] = jnp.zeros_like(acc)

    sfull = jnp.concatenate([lo_ref[0], hi_ref[0]], axis=1)
    h2 = jnp.maximum(sfull * rdeg_ref[...] + b2_ref[...] + p1_ref[...], 0.0)
    h2e = jnp.concatenate([h2, jnp.ones((BN, 1), _F32)], axis=1)
    oh = (batch_ref[...] == lax.broadcasted_iota(_I32, (BN, B), 1)).astype(_F32)
    acc[...] += lax.dot_general(oh, h2e, (((0,), (0,)), ((), ())),
                                preferred_element_type=_F32)

    @pl.when(i == NB - 1)
    def _():
        sums = acc[:, :HID]
        cnt = jnp.maximum(acc[:, HID:HID + 1], 1.0)
        hg = sums / cnt
        out_ref[...] = _matT(hg, clsw_ref[...]) + clsb_ref[...]


def _full_spec(shape):
    return pl.BlockSpec(shape, lambda i: tuple(0 for _ in shape))


def _row_spec(width):
    return pl.BlockSpec((BN, width), lambda i: (i, 0))


def _part_spec(p):
    return pl.BlockSpec((1, BN, 16), lambda i, _p=p: (_p, i, 0))


def kernel(x, edge_index, batch, shape_emb, color_emb,
           g1_wl, g1_bl, g1_wr, g2_wl, g2_bl, g2_wr, cls_w, cls_b):
    x = x.astype(_I32)
    edge_index = edge_index.astype(_I32)
    batch = batch.astype(_I32)

    x_p = jnp.concatenate([x, jnp.zeros((NPAD - N, 2), _I32)], axis=0)
    src_p = jnp.concatenate(
        [edge_index[0], jnp.zeros((EP - E,), _I32)]).reshape(ER, SUP, CH)
    dst_p = jnp.concatenate(
        [edge_index[1], jnp.full((EP - E,), N, _I32)]).reshape(ER, SUP, CH)
    batch_p = jnp.concatenate([batch, jnp.full((NPAD - N,), B, _I32)])
    batch_p = batch_p.reshape(NPAD, 1)
    zc = jnp.zeros((ROWS_PT, 16), _F32)
    zd = jnp.zeros((ROWS_PT,), _F32)
    b1 = g1_bl.reshape(1, HID)
    b2 = g2_bl.reshape(1, HID)
    cb = cls_b.reshape(1, NUM_CLASSES)

    m0, p0 = pl.pallas_call(
        _prep_body,
        grid=(NB,),
        in_specs=[_row_spec(2), _full_spec((16, EMB)), _full_spec((16, EMB)),
                  _full_spec((HID, 2 * EMB)), _full_spec((HID, 2 * EMB))],
        out_specs=[pl.BlockSpec((2, BN, 16), lambda i: (0, i, 0)),
                   _row_spec(HID)],
        out_shape=[jax.ShapeDtypeStruct((2, NPAD, 16), _F32),
                   jax.ShapeDtypeStruct((NPAD, HID), _F32)],
    )(x_p, shape_emb, color_emb, g1_wl, g1_wr)

    s1, deg2 = _make_edge_pass(True)(src_p, dst_p, m0, zc, zd)
    degt = jnp.transpose(deg2)

    m1, p1, rdeg = pl.pallas_call(
        _mid_body,
        grid=(NB,),
        in_specs=[_part_spec(0), _part_spec(1), _row_spec(2), _row_spec(HID),
                  _full_spec((1, HID)), _full_spec((HID, HID)),
                  _full_spec((HID, HID))],
        out_specs=[pl.BlockSpec((2, BN, 16), lambda i: (0, i, 0)),
                   _row_spec(HID), _row_spec(1)],
        out_shape=[jax.ShapeDtypeStruct((2, NPAD, 16), _F32),
                   jax.ShapeDtypeStruct((NPAD, HID), _F32),
                   jax.ShapeDtypeStruct((NPAD, 1), _F32)],
    )(s1, s1, degt, p0, b1, g2_wl, g2_wr)

    s2 = _make_edge_pass(False)(src_p, dst_p, m1, zc)
    if isinstance(s2, (tuple, list)):
        s2 = s2[0]

    out = pl.pallas_call(
        _final_body,
        grid=(NB,),
        in_specs=[_part_spec(0), _part_spec(1), _row_spec(1), _row_spec(HID),
                  _full_spec((1, HID)), _row_spec(1),
                  _full_spec((NUM_CLASSES, HID)), _full_spec((1, NUM_CLASSES))],
        out_specs=pl.BlockSpec((B, NUM_CLASSES), lambda i: (0, 0)),
        out_shape=jax.ShapeDtypeStruct((B, NUM_CLASSES), _F32),
        scratch_shapes=[pltpu.VMEM((B, HID + 1), _F32)],
    )(s2, s2, rdeg, p1, b2, batch_p, cls_w, cb)
    return out


# speedup vs baseline: 1.1354x; 1.1354x over previous
import functools

import jax
import jax.numpy as jnp
from jax import lax
from jax.experimental import pallas as pl
from jax.experimental.pallas import tpu as pltpu
from jax.experimental.pallas import tpu_sc as plsc

N = 100000
E = 1600000
B = 128
NUM_SHAPES = 16
NUM_COLORS = 16
NUM_CLASSES = 8
EMB = 16
HID = 32

BN = 1024
NPAD = 100352
NB = NPAD // BN
ROWS_PT = NPAD // 16

NSUB = 16
CH = 128
SUP = 8
NSUP = 98
EPT = NSUP * SUP * CH
EP = EPT * NSUB
ER = EP // (SUP * CH)

_F32 = jnp.float32
_I32 = jnp.int32


def _make_edge_pass(with_deg: bool):
    outs = [jax.ShapeDtypeStruct((2, NPAD, 16), _F32)]
    scratch = [
        pltpu.VMEM((3, SUP, CH), _I32),
        pltpu.VMEM((3, SUP, CH), _I32),
        pltpu.VMEM((SUP, CH, 16), _F32),
        pltpu.VMEM_SHARED((NPAD, 16), _F32),
        pltpu.SemaphoreType.DMA,
    ] + [pltpu.SemaphoreType.DMA] * SUP
    if with_deg:
        outs.append(jax.ShapeDtypeStruct((2, NPAD), _F32))
        scratch += [
            pltpu.VMEM((CH,), _F32),
            pltpu.VMEM_SHARED((NPAD,), _F32),
            pltpu.SemaphoreType.DMA,
        ]

    def body(*refs):
        if with_deg:
            (src_h, dst_h, m_h, zc_h, zd_h, s_out, deg_out,
             srcb, dstb, rows, acc_sh, sem_g, *rest) = refs
            sem_s = rest[:SUP]
            ones_v, deg_sh, sem_d = rest[SUP:]
        else:
            (src_h, dst_h, m_h, zc_h, s_out,
             srcb, dstb, rows, acc_sh, sem_g, *sem_s) = refs
        c = lax.axis_index("c")
        s = lax.axis_index("s")
        r0 = s * ROWS_PT

        pltpu.sync_copy(zc_h, acc_sh.at[pl.ds(r0, ROWS_PT)])
        if with_deg:
            pltpu.sync_copy(zd_h, deg_sh.at[pl.ds(r0, ROWS_PT)])
            for j in range(CH // 16):
                ones_v[pl.ds(j * 16, 16)] = jnp.ones((16,), _F32)
        plsc.subcore_barrier()

        crow0 = s * NSUP
        mt = m_h.at[c]
        pltpu.sync_copy(src_h.at[crow0], srcb.at[0])
        pltpu.sync_copy(dst_h.at[crow0], dstb.at[0])

        def sup_body(sup, carry):
            b = lax.rem(sup, 3)
            gs = []
            for j in range(SUP):
                @pl.when(sup > 0)
                def _(j=j):
                    pltpu.make_async_copy(mt.at[pl.ds(j * CH, CH)],
                                          rows.at[j], sem_s[j]).wait()
                gs.append(pltpu.async_copy(mt.at[srcb.at[b, j]], rows.at[j],
                                           sem_g))

            if with_deg:
                @pl.when(sup > 0)
                def _():
                    for _ in range(SUP // 2):
                        pltpu.make_async_copy(zd_h.at[pl.ds(0, CH)],
                                              ones_v, sem_d).wait()

                @pl.when(c == 0)
                def _():
                    for j in range(SUP // 2):
                        pltpu.async_copy(ones_v, deg_sh.at[dstb.at[b, j]],
                                         sem_d, add=True)

                @pl.when(c != 0)
                def _():
                    for j in range(SUP // 2, SUP):
                        pltpu.async_copy(ones_v, deg_sh.at[dstb.at[b, j]],
                                         sem_d, add=True)

            @pl.when(sup + 1 < NSUP)
            def _():
                nb = lax.rem(sup + 1, 3)
                nrow = crow0 + sup + 1
                pltpu.sync_copy(src_h.at[nrow], srcb.at[nb])
                pltpu.sync_copy(dst_h.at[nrow], dstb.at[nb])

            for j in range(SUP):
                gs[j].wait()
                pltpu.async_copy(rows.at[j], acc_sh.at[dstb.at[b, j]],
                                 sem_s[j], add=True)
            return carry

        lax.fori_loop(0, NSUP, sup_body, 0)
        for j in range(SUP):
            pltpu.make_async_copy(mt.at[pl.ds(j * CH, CH)],
                                  rows.at[j], sem_s[j]).wait()
        if with_deg:
            for _ in range(SUP // 2):
                pltpu.make_async_copy(zd_h.at[pl.ds(0, CH)],
                                      ones_v, sem_d).wait()

        plsc.subcore_barrier()
        pltpu.sync_copy(acc_sh.at[pl.ds(r0, ROWS_PT)],
                        s_out.at[c].at[pl.ds(r0, ROWS_PT)])
        if with_deg:
            pltpu.sync_copy(deg_sh.at[pl.ds(r0, ROWS_PT)],
                            deg_out.at[c].at[pl.ds(r0, ROWS_PT)])

    mesh = plsc.VectorSubcoreMesh(core_axis_name="c", subcore_axis_name="s")
    return pl.kernel(body, out_type=tuple(outs), mesh=mesh,
                     scratch_types=scratch,
                     compiler_params=pltpu.CompilerParams(
                         use_tc_tiling_on_sc=False))


def _oh16(idx_col):
    return (idx_col == lax.broadcasted_iota(_I32, (BN, 16), 1)).astype(_F32)


def _matT(a, w):
    return lax.dot_general(a, w, (((1,), (1,)), ((), ())),
                           preferred_element_type=_F32)


def _prep_body(x_ref, se_ref, ce_ref, w1l_ref, w1r_ref, m_ref, p_ref):
    ohs = _oh16(x_ref[:, 0:1])
    ohc = _oh16(x_ref[:, 1:2])
    h0 = jnp.concatenate(
        [jnp.dot(ohs, se_ref[...], preferred_element_type=_F32),
         jnp.dot(ohc, ce_ref[...], preferred_element_type=_F32)], axis=1)
    m = _matT(h0, w1l_ref[...])
    m_ref[0] = m[:, :16]
    m_ref[1] = m[:, 16:]
    p_ref[...] = _matT(h0, w1r_ref[...])


def _mid_body(lo_ref, hi_ref, degt_ref, p0_ref, b1_ref, w2l_ref, w2r_ref,
              m_ref, p1_ref, rdeg_ref):
    deg = degt_ref[:, 0:1] + degt_ref[:, 1:2]
    rdeg = 1.0 / jnp.maximum(deg, 1.0)
    sfull = jnp.concatenate([lo_ref[0], hi_ref[0]], axis=1)
    h1 = jnp.maximum(sfull * rdeg + b1_ref[...] + p0_ref[...], 0.0)
    m1 = _matT(h1, w2l_ref[...])
    m_ref[0] = m1[:, :16]
    m_ref[1] = m1[:, 16:]
    p1_ref[...] = _matT(h1, w2r_ref[...])
    rdeg_ref[...] = rdeg


def _final_body(lo_ref, hi_ref, rdeg_ref, p1_ref, b2_ref, batch_ref,
                clsw_ref, clsb_ref, out_ref, acc):
    i = pl.program_id(0)

    @pl.when(i == 0)
    def _():
        acc[...] = jnp.zeros_like(acc)

    sfull = jnp.concatenate([lo_ref[0], hi_ref[0]], axis=1)
    h2 = jnp.maximum(sfull * rdeg_ref[...] + b2_ref[...] + p1_ref[...], 0.0)
    h2e = jnp.concatenate([h2, jnp.ones((BN, 1), _F32)], axis=1)
    oh = (batch_ref[...] == lax.broadcasted_iota(_I32, (BN, B), 1)).astype(_F32)
    acc[...] += lax.dot_general(oh, h2e, (((0,), (0,)), ((), ())),
                                preferred_element_type=_F32)

    @pl.when(i == NB - 1)
    def _():
        sums = acc[:, :HID]
        cnt = jnp.maximum(acc[:, HID:HID + 1], 1.0)
        hg = sums / cnt
        out_ref[...] = _matT(hg, clsw_ref[...]) + clsb_ref[...]


def _full_spec(shape):
    return pl.BlockSpec(shape, lambda i: tuple(0 for _ in shape))


def _row_spec(width):
    return pl.BlockSpec((BN, width), lambda i: (i, 0))


def _part_spec(p):
    return pl.BlockSpec((1, BN, 16), lambda i, _p=p: (_p, i, 0))


def kernel(x, edge_index, batch, shape_emb, color_emb,
           g1_wl, g1_bl, g1_wr, g2_wl, g2_bl, g2_wr, cls_w, cls_b):
    x = x.astype(_I32)
    edge_index = edge_index.astype(_I32)
    batch = batch.astype(_I32)

    x_p = jnp.concatenate([x, jnp.zeros((NPAD - N, 2), _I32)], axis=0)
    src_p = jnp.concatenate(
        [edge_index[0], jnp.zeros((EP - E,), _I32)]).reshape(ER, SUP, CH)
    dst_p = jnp.concatenate(
        [edge_index[1], jnp.full((EP - E,), N, _I32)]).reshape(ER, SUP, CH)
    batch_p = jnp.concatenate([batch, jnp.full((NPAD - N,), B, _I32)])
    batch_p = batch_p.reshape(NPAD, 1)
    zc = jnp.zeros((ROWS_PT, 16), _F32)
    zd = jnp.zeros((ROWS_PT,), _F32)
    b1 = g1_bl.reshape(1, HID)
    b2 = g2_bl.reshape(1, HID)
    cb = cls_b.reshape(1, NUM_CLASSES)

    m0, p0 = pl.pallas_call(
        _prep_body,
        grid=(NB,),
        in_specs=[_row_spec(2), _full_spec((16, EMB)), _full_spec((16, EMB)),
                  _full_spec((HID, 2 * EMB)), _full_spec((HID, 2 * EMB))],
        out_specs=[pl.BlockSpec((2, BN, 16), lambda i: (0, i, 0)),
                   _row_spec(HID)],
        out_shape=[jax.ShapeDtypeStruct((2, NPAD, 16), _F32),
                   jax.ShapeDtypeStruct((NPAD, HID), _F32)],
    )(x_p, shape_emb, color_emb, g1_wl, g1_wr)

    s1, deg2 = _make_edge_pass(True)(src_p, dst_p, m0, zc, zd)
    degt = jnp.transpose(deg2)

    m1, p1, rdeg = pl.pallas_call(
        _mid_body,
        grid=(NB,),
        in_specs=[_part_spec(0), _part_spec(1), _row_spec(2), _row_spec(HID),
                  _full_spec((1, HID)), _full_spec((HID, HID)),
                  _full_spec((HID, HID))],
        out_specs=[pl.BlockSpec((2, BN, 16), lambda i: (0, i, 0)),
                   _row_spec(HID), _row_spec(1)],
        out_shape=[jax.ShapeDtypeStruct((2, NPAD, 16), _F32),
                   jax.ShapeDtypeStruct((NPAD, HID), _F32),
                   jax.ShapeDtypeStruct((NPAD, 1), _F32)],
    )(s1, s1, degt, p0, b1, g2_wl, g2_wr)

    s2 = _make_edge_pass(False)(src_p, dst_p, m1, zc)
    if isinstance(s2, (tuple, list)):
        s2 = s2[0]

    out = pl.pallas_call(
        _final_body,
        grid=(NB,),
        in_specs=[_part_spec(0), _part_spec(1), _row_spec(1), _row_spec(HID),
                  _full_spec((1, HID)), _row_spec(1),
                  _full_spec((NUM_CLASSES, HID)), _full_spec((1, NUM_CLASSES))],
        out_specs=pl.BlockSpec((B, NUM_CLASSES), lambda i: (0, 0)),
        out_shape=jax.ShapeDtypeStruct((B, NUM_CLASSES), _F32),
        scratch_shapes=[pltpu.VMEM((B, HID + 1), _F32)],
    )(s2, s2, rdeg, p1, b2, batch_p, cls_w, cb)
    return out


# speedup vs baseline: 1.2439x; 1.0956x over previous
import functools

import jax
import jax.numpy as jnp
from jax import lax
from jax.experimental import pallas as pl
from jax.experimental.pallas import tpu as pltpu
from jax.experimental.pallas import tpu_sc as plsc

N = 100000
E = 1600000
B = 128
NUM_SHAPES = 16
NUM_COLORS = 16
NUM_CLASSES = 8
EMB = 16
HID = 32

BN = 3584
NPAD = 100352
NB = NPAD // BN
ROWS_PT = NPAD // 16

NSUB = 16
CH = 128
SUP = 8
NSUP = 98
EPT = NSUP * SUP * CH
EP = EPT * NSUB
ER = EP // (SUP * CH)

_F32 = jnp.float32
_I32 = jnp.int32


def _make_edge_pass(with_deg: bool):
    outs = [jax.ShapeDtypeStruct((2, NPAD, 16), _F32)]
    scratch = [
        pltpu.VMEM((3, SUP, CH), _I32),
        pltpu.VMEM((3, SUP, CH), _I32),
        pltpu.VMEM((SUP, CH, 16), _F32),
        pltpu.VMEM_SHARED((NPAD, 16), _F32),
        pltpu.SemaphoreType.DMA,
    ] + [pltpu.SemaphoreType.DMA] * SUP
    if with_deg:
        outs.append(jax.ShapeDtypeStruct((2, NPAD), _F32))
        scratch += [
            pltpu.VMEM((CH,), _F32),
            pltpu.VMEM_SHARED((NPAD,), _F32),
            pltpu.SemaphoreType.DMA,
        ]

    def body(*refs):
        if with_deg:
            (src_h, dst_h, m_h, zc_h, zd_h, s_out, deg_out,
             srcb, dstb, rows, acc_sh, sem_g, *rest) = refs
            sem_s = rest[:SUP]
            ones_v, deg_sh, sem_d = rest[SUP:]
        else:
            (src_h, dst_h, m_h, zc_h, s_out,
             srcb, dstb, rows, acc_sh, sem_g, *sem_s) = refs
        c = lax.axis_index("c")
        s = lax.axis_index("s")
        r0 = s * ROWS_PT

        pltpu.sync_copy(zc_h, acc_sh.at[pl.ds(r0, ROWS_PT)])
        if with_deg:
            pltpu.sync_copy(zd_h, deg_sh.at[pl.ds(r0, ROWS_PT)])
            for j in range(CH // 16):
                ones_v[pl.ds(j * 16, 16)] = jnp.ones((16,), _F32)
        plsc.subcore_barrier()

        crow0 = s * NSUP
        mt = m_h.at[c]
        pltpu.sync_copy(src_h.at[crow0], srcb.at[0])
        pltpu.sync_copy(dst_h.at[crow0], dstb.at[0])

        def sup_body(sup, carry):
            b = lax.rem(sup, 3)
            gs = []
            for j in range(SUP):
                @pl.when(sup > 0)
                def _(j=j):
                    pltpu.make_async_copy(mt.at[pl.ds(j * CH, CH)],
                                          rows.at[j], sem_s[j]).wait()
                gs.append(pltpu.async_copy(mt.at[srcb.at[b, j]], rows.at[j],
                                           sem_g))

            if with_deg:
                @pl.when(sup > 0)
                def _():
                    for _ in range(SUP // 2):
                        pltpu.make_async_copy(zd_h.at[pl.ds(0, CH)],
                                              ones_v, sem_d).wait()

                @pl.when(c == 0)
                def _():
                    for j in range(SUP // 2):
                        pltpu.async_copy(ones_v, deg_sh.at[dstb.at[b, j]],
                                         sem_d, add=True)

                @pl.when(c != 0)
                def _():
                    for j in range(SUP // 2, SUP):
                        pltpu.async_copy(ones_v, deg_sh.at[dstb.at[b, j]],
                                         sem_d, add=True)

            @pl.when(sup + 1 < NSUP)
            def _():
                nb = lax.rem(sup + 1, 3)
                nrow = crow0 + sup + 1
                pltpu.sync_copy(src_h.at[nrow], srcb.at[nb])
                pltpu.sync_copy(dst_h.at[nrow], dstb.at[nb])

            for j in range(SUP):
                gs[j].wait()
                pltpu.async_copy(rows.at[j], acc_sh.at[dstb.at[b, j]],
                                 sem_s[j], add=True)
            return carry

        lax.fori_loop(0, NSUP, sup_body, 0)
        for j in range(SUP):
            pltpu.make_async_copy(mt.at[pl.ds(j * CH, CH)],
                                  rows.at[j], sem_s[j]).wait()
        if with_deg:
            for _ in range(SUP // 2):
                pltpu.make_async_copy(zd_h.at[pl.ds(0, CH)],
                                      ones_v, sem_d).wait()

        plsc.subcore_barrier()
        pltpu.sync_copy(acc_sh.at[pl.ds(r0, ROWS_PT)],
                        s_out.at[c].at[pl.ds(r0, ROWS_PT)])
        if with_deg:
            pltpu.sync_copy(deg_sh.at[pl.ds(r0, ROWS_PT)],
                            deg_out.at[c].at[pl.ds(r0, ROWS_PT)])

    mesh = plsc.VectorSubcoreMesh(core_axis_name="c", subcore_axis_name="s")
    return pl.kernel(body, out_type=tuple(outs), mesh=mesh,
                     scratch_types=scratch,
                     compiler_params=pltpu.CompilerParams(
                         use_tc_tiling_on_sc=False))


def _oh16(idx_col):
    return (idx_col == lax.broadcasted_iota(_I32, (BN, 16), 1)).astype(_F32)


def _matT(a, w):
    return lax.dot_general(a, w, (((1,), (1,)), ((), ())),
                           preferred_element_type=_F32)


def _prep_body(x_ref, se_ref, ce_ref, w1l_ref, w1r_ref, m_ref, p_ref):
    ohs = _oh16(x_ref[:, 0:1])
    ohc = _oh16(x_ref[:, 1:2])
    h0 = jnp.concatenate(
        [jnp.dot(ohs, se_ref[...], preferred_element_type=_F32),
         jnp.dot(ohc, ce_ref[...], preferred_element_type=_F32)], axis=1)
    m = _matT(h0, w1l_ref[...])
    m_ref[0] = m[:, :16]
    m_ref[1] = m[:, 16:]
    p_ref[...] = _matT(h0, w1r_ref[...])


def _mid_body(lo_ref, hi_ref, degt_ref, p0_ref, b1_ref, w2l_ref, w2r_ref,
              m_ref, p1_ref, rdeg_ref):
    deg = degt_ref[:, 0:1] + degt_ref[:, 1:2]
    rdeg = 1.0 / jnp.maximum(deg, 1.0)
    sfull = jnp.concatenate([lo_ref[0], hi_ref[0]], axis=1)
    h1 = jnp.maximum(sfull * rdeg + b1_ref[...] + p0_ref[...], 0.0)
    m1 = _matT(h1, w2l_ref[...])
    m_ref[0] = m1[:, :16]
    m_ref[1] = m1[:, 16:]
    p1_ref[...] = _matT(h1, w2r_ref[...])
    rdeg_ref[...] = rdeg


def _final_body(lo_ref, hi_ref, rdeg_ref, p1_ref, b2_ref, batch_ref,
                clsw_ref, clsb_ref, out_ref, acc):
    i = pl.program_id(0)

    @pl.when(i == 0)
    def _():
        acc[...] = jnp.zeros_like(acc)

    sfull = jnp.concatenate([lo_ref[0], hi_ref[0]], axis=1)
    h2 = jnp.maximum(sfull * rdeg_ref[...] + b2_ref[...] + p1_ref[...], 0.0)
    h2e = jnp.concatenate([h2, jnp.ones((BN, 1), _F32)], axis=1)
    oh = (batch_ref[...] == lax.broadcasted_iota(_I32, (BN, B), 1)).astype(_F32)
    acc[...] += lax.dot_general(oh, h2e, (((0,), (0,)), ((), ())),
                                preferred_element_type=_F32)

    @pl.when(i == NB - 1)
    def _():
        sums = acc[:, :HID]
        cnt = jnp.maximum(acc[:, HID:HID + 1], 1.0)
        hg = sums / cnt
        out_ref[...] = _matT(hg, clsw_ref[...]) + clsb_ref[...]


def _full_spec(shape):
    return pl.BlockSpec(shape, lambda i: tuple(0 for _ in shape))


def _row_spec(width):
    return pl.BlockSpec((BN, width), lambda i: (i, 0))


def _part_spec(p):
    return pl.BlockSpec((1, BN, 16), lambda i, _p=p: (_p, i, 0))


def kernel(x, edge_index, batch, shape_emb, color_emb,
           g1_wl, g1_bl, g1_wr, g2_wl, g2_bl, g2_wr, cls_w, cls_b):
    x = x.astype(_I32)
    edge_index = edge_index.astype(_I32)
    batch = batch.astype(_I32)

    x_p = jnp.concatenate([x, jnp.zeros((NPAD - N, 2), _I32)], axis=0)
    src_p = jnp.concatenate(
        [edge_index[0], jnp.zeros((EP - E,), _I32)]).reshape(ER, SUP, CH)
    dst_p = jnp.concatenate(
        [edge_index[1], jnp.full((EP - E,), N, _I32)]).reshape(ER, SUP, CH)
    batch_p = jnp.concatenate([batch, jnp.full((NPAD - N,), B, _I32)])
    batch_p = batch_p.reshape(NPAD, 1)
    zc = jnp.zeros((ROWS_PT, 16), _F32)
    zd = jnp.zeros((ROWS_PT,), _F32)
    b1 = g1_bl.reshape(1, HID)
    b2 = g2_bl.reshape(1, HID)
    cb = cls_b.reshape(1, NUM_CLASSES)

    m0, p0 = pl.pallas_call(
        _prep_body,
        grid=(NB,),
        in_specs=[_row_spec(2), _full_spec((16, EMB)), _full_spec((16, EMB)),
                  _full_spec((HID, 2 * EMB)), _full_spec((HID, 2 * EMB))],
        out_specs=[pl.BlockSpec((2, BN, 16), lambda i: (0, i, 0)),
                   _row_spec(HID)],
        out_shape=[jax.ShapeDtypeStruct((2, NPAD, 16), _F32),
                   jax.ShapeDtypeStruct((NPAD, HID), _F32)],
    )(x_p, shape_emb, color_emb, g1_wl, g1_wr)

    s1, deg2 = _make_edge_pass(True)(src_p, dst_p, m0, zc, zd)
    degt = jnp.transpose(deg2)

    m1, p1, rdeg = pl.pallas_call(
        _mid_body,
        grid=(NB,),
        in_specs=[_part_spec(0), _part_spec(1), _row_spec(2), _row_spec(HID),
                  _full_spec((1, HID)), _full_spec((HID, HID)),
                  _full_spec((HID, HID))],
        out_specs=[pl.BlockSpec((2, BN, 16), lambda i: (0, i, 0)),
                   _row_spec(HID), _row_spec(1)],
        out_shape=[jax.ShapeDtypeStruct((2, NPAD, 16), _F32),
                   jax.ShapeDtypeStruct((NPAD, HID), _F32),
                   jax.ShapeDtypeStruct((NPAD, 1), _F32)],
    )(s1, s1, degt, p0, b1, g2_wl, g2_wr)

    s2 = _make_edge_pass(False)(src_p, dst_p, m1, zc)
    if isinstance(s2, (tuple, list)):
        s2 = s2[0]

    out = pl.pallas_call(
        _final_body,
        grid=(NB,),
        in_specs=[_part_spec(0), _part_spec(1), _row_spec(1), _row_spec(HID),
                  _full_spec((1, HID)), _row_spec(1),
                  _full_spec((NUM_CLASSES, HID)), _full_spec((1, NUM_CLASSES))],
        out_specs=pl.BlockSpec((B, NUM_CLASSES), lambda i: (0, 0)),
        out_shape=jax.ShapeDtypeStruct((B, NUM_CLASSES), _F32),
        scratch_shapes=[pltpu.VMEM((B, HID + 1), _F32)],
    )(s2, s2, rdeg, p1, b2, batch_p, cls_w, cb)
    return out
